# baseline (speedup 1.0000x reference)
import functools

import jax
import jax.numpy as jnp
from jax import lax
from jax.experimental import pallas as pl
from jax.experimental.pallas import tpu as pltpu
from jax.experimental.pallas import tpu_sc as plsc

L = 16


@functools.lru_cache(maxsize=None)
def _build_sc_row_gather(n_rows: int, n_pix: int, d: int):
    info = plsc.get_sparse_core_info()
    nc, ns = info.num_cores, info.num_subcores
    nw = nc * ns
    assert n_rows % (nw * n_pix) == 0

    rows_per_w = n_rows // nw
    tile_r = 64
    nbuf = 2
    n_tiles = rows_per_w // tile_r
    n_rounds = n_tiles // nbuf
    assert n_tiles % nbuf == 0 and n_pix % tile_r == 0

    mesh = plsc.VectorSubcoreMesh(core_axis_name="c", subcore_axis_name="s")

    @functools.partial(
        pl.kernel,
        mesh=mesh,
        out_type=jax.ShapeDtypeStruct((n_rows, d), jnp.float32),
        scratch_types=[
            pltpu.VMEM((n_pix,), jnp.int32),
            pltpu.VMEM((nbuf * tile_r,), jnp.int32),
            pltpu.VMEM((nbuf, tile_r, d), jnp.float32),
        ]
        + [pltpu.SemaphoreType.DMA] * (2 * nbuf),
        compiler_params=pltpu.CompilerParams(
            disable_bounds_checks=True, disable_semaphore_checks=True
        ),
    )
    def sc_row_gather(x_hbm, perm_hbm, out_hbm, perm_v, idx_v, rows_v, *sems):
        gsem, ssem = sems[:nbuf], sems[nbuf:]
        wid = lax.axis_index("s") * nc + lax.axis_index("c")
        base = wid * rows_per_w
        pltpu.sync_copy(perm_hbm, perm_v)

        def fill_idx(t, slot):
            g0 = base + t * tile_r
            boff = (g0 // n_pix) * n_pix
            p0 = g0 % n_pix
            for k in range(tile_r // L):
                idx_v[pl.ds(slot * tile_r + k * L, L)] = (
                    perm_v[pl.ds(p0 + k * L, L)] + boff
                )

        def gather(t, slot):
            return pltpu.make_async_copy(
                x_hbm.at[idx_v.at[pl.ds(slot * tile_r, tile_r)]],
                rows_v.at[slot],
                gsem[slot],
            )

        def store(t, slot):
            return pltpu.make_async_copy(
                rows_v.at[slot],
                out_hbm.at[pl.ds(base + t * tile_r, tile_r)],
                ssem[slot],
            )

        for s in range(nbuf - 1):
            fill_idx(s, s)
            gather(s, s).start()

        def round_body(r, carry):
            t0 = r * nbuf
            for s in range(nbuf):
                t = t0 + s
                gather(t, s).wait()
                store(t, s).start()

                @pl.when(t >= 1)
                def _():
                    store(t - 1, (s - 1) % nbuf).wait()

                tn = t + nbuf - 1
                sn = (s + nbuf - 1) % nbuf

                @pl.when(tn < n_tiles)
                def _():
                    fill_idx(tn, sn)
                    gather(tn, sn).start()

            return carry

        lax.fori_loop(0, n_rounds, round_body, 0)
        store(n_tiles - 1, (n_tiles - 1) % nbuf).wait()

    return sc_row_gather


def kernel(x, perm):
    b, c, h, w = x.shape
    n_pix = h * w
    xt = x.transpose(0, 2, 3, 1).reshape(b * n_pix, c)
    out = _build_sc_row_gather(b * n_pix, n_pix, c)(xt, perm)
    return out.reshape(b, h, w, c).transpose(0, 3, 1, 2)


# speedup vs baseline: 1.0068x; 1.0068x over previous
import functools

import jax
import jax.numpy as jnp
from jax import lax
from jax.experimental import pallas as pl
from jax.experimental.pallas import tpu as pltpu
from jax.experimental.pallas import tpu_sc as plsc

L = 16


@functools.lru_cache(maxsize=None)
def _build_sc_row_gather(n_rows: int, n_pix: int, d: int):
    info = plsc.get_sparse_core_info()
    nc, ns = info.num_cores, info.num_subcores
    nw = nc * ns
    assert n_rows % (nw * n_pix) == 0

    rows_per_w = n_rows // nw
    tile_r = 64
    nbuf = 2
    n_tiles = rows_per_w // tile_r
    n_rounds = n_tiles // nbuf
    assert n_tiles % nbuf == 0 and n_pix % tile_r == 0

    mesh = plsc.VectorSubcoreMesh(core_axis_name="c", subcore_axis_name="s")

    @functools.partial(
        pl.kernel,
        mesh=mesh,
        out_type=jax.ShapeDtypeStruct((n_rows, d), jnp.float32),
        scratch_types=[
            pltpu.VMEM((n_pix,), jnp.int32),
            pltpu.VMEM((nbuf * tile_r,), jnp.int32),
            pltpu.VMEM((nbuf, tile_r, d), jnp.float32),
        ]
        + [pltpu.SemaphoreType.DMA] * (2 * nbuf),
    )
    def sc_row_gather(x_hbm, perm_hbm, out_hbm, perm_v, idx_v, rows_v, *sems):
        gsem, ssem = sems[:nbuf], sems[nbuf:]
        wid = lax.axis_index("s") * nc + lax.axis_index("c")
        base = wid * rows_per_w
        pltpu.sync_copy(perm_hbm, perm_v)

        def fill_idx(t, slot):
            g0 = base + t * tile_r
            boff = (g0 // n_pix) * n_pix
            p0 = g0 % n_pix
            for k in range(tile_r // L):
                idx_v[pl.ds(slot * tile_r + k * L, L)] = (
                    perm_v[pl.ds(p0 + k * L, L)] + boff
                )

        def gather(t, slot):
            return pltpu.make_async_copy(
                x_hbm.at[idx_v.at[pl.ds(slot * tile_r, tile_r)]],
                rows_v.at[slot],
                gsem[slot],
            )

        def store(t, slot):
            return pltpu.make_async_copy(
                rows_v.at[slot],
                out_hbm.at[pl.ds(base + t * tile_r, tile_r)],
                ssem[slot],
            )

        for s in range(nbuf - 1):
            fill_idx(s, s)
            gather(s, s).start()

        def round_body(r, carry):
            t0 = r * nbuf
            for s in range(nbuf):
                t = t0 + s
                gather(t, s).wait()
                store(t, s).start()

                @pl.when(t >= 1)
                def _():
                    store(t - 1, (s - 1) % nbuf).wait()

                tn = t + nbuf - 1
                sn = (s + nbuf - 1) % nbuf

                @pl.when(tn < n_tiles)
                def _():
                    fill_idx(tn, sn)
                    gather(tn, sn).start()

            return carry

        lax.fori_loop(0, n_rounds, round_body, 0)
        store(n_tiles - 1, (n_tiles - 1) % nbuf).wait()

    return sc_row_gather


def kernel(x, perm):
    b, c, h, w = x.shape
    n_pix = h * w
    xt = x.transpose(0, 2, 3, 1).reshape(b * n_pix, c)
    out = _build_sc_row_gather(b * n_pix, n_pix, c)(xt, perm)
    return out.reshape(b, h, w, c).transpose(0, 3, 1, 2)
